# baseline (speedup 1.0000x reference)
import functools

import jax
import jax.numpy as jnp
from jax import lax
from jax.experimental import pallas as pl
from jax.experimental.pallas import tpu as pltpu
from jax.experimental.pallas import tpu_sc as plsc

_B = 512
_C = 2048
_N = 4096
_NBITS = 14
_M = 1 << _NBITS

_BN = 512

_NW = 32
_TOT = _B * _N
_PW = _TOT // _NW
_CH = 8192
_NCH = _PW // _CH
_SUB = 128
_NSUB = _CH // _SUB


def _addr_body(x_ref, w_ref, out_ref):
    w = w_ref[...]
    whi = jnp.floor(w * (1.0 / 128.0))
    wlo = w - whi * 128.0
    dims = (((1,), (1,)), ((), ()))
    lo = lax.dot_general(wlo.astype(jnp.bfloat16), x_ref[...], dims,
                         preferred_element_type=jnp.float32)
    hi = lax.dot_general(whi.astype(jnp.bfloat16), x_ref[...], dims,
                         preferred_element_type=jnp.float32)
    n0 = pl.program_id(0) * _BN
    n = lax.broadcasted_iota(jnp.int32, lo.shape, 0) + n0
    a = lo.astype(jnp.int32) + (hi.astype(jnp.int32) << 7)
    out_ref[...] = (((n >> 3) << 17) + ((a >> 7) << 10)
                    + ((n & 7) << 7) + (a & 127))


_addr_call = pl.pallas_call(
    _addr_body,
    grid=(_N // _BN,),
    in_specs=[
        pl.BlockSpec((_B, _C), lambda j: (0, 0)),
        pl.BlockSpec((_BN, _C), lambda j: (j, 0)),
    ],
    out_specs=pl.BlockSpec((_BN, _B), lambda j: (j, 0)),
    out_shape=jax.ShapeDtypeStruct((_N, _B), jnp.int32),
)


def _lookup_body(idx_hbm, mem_hbm, out_hbm, idx_v, val_v, sem):
    wid = lax.axis_index("s") * 2 + lax.axis_index("c")
    base = wid * _PW

    def outer(i, carry):
        o0 = base + i * _CH
        pltpu.sync_copy(idx_hbm.at[pl.ds(o0, _CH)], idx_v)
        copies = [
            pltpu.async_copy(
                mem_hbm.at[idx_v.at[pl.ds(k * _SUB, _SUB)]],
                val_v.at[pl.ds(k * _SUB, _SUB)],
                sem,
            )
            for k in range(_NSUB)
        ]
        for c in copies:
            c.wait()

        def inner(j, c):
            v = val_v[pl.ds(j * 16, 16)]
            one = jnp.full((16,), 1, jnp.int32)
            zero = jnp.zeros((16,), jnp.int32)
            val_v[pl.ds(j * 16, 16)] = jnp.where(v == one, one, zero)
            return c

        lax.fori_loop(0, _CH // 16, inner, 0, unroll=8)
        pltpu.sync_copy(val_v, out_hbm.at[pl.ds(o0, _CH)])
        return carry

    lax.fori_loop(0, _NCH, outer, 0)


_lookup_call = functools.partial(
    pl.kernel,
    mesh=plsc.VectorSubcoreMesh(core_axis_name="c", subcore_axis_name="s"),
    out_type=jax.ShapeDtypeStruct((_TOT,), jnp.int32),
    scratch_types=[
        pltpu.VMEM((_CH,), jnp.int32),
        pltpu.VMEM((_CH,), jnp.int32),
        pltpu.SemaphoreType.DMA,
    ],
)(_lookup_body)


def kernel(input_bits, connections, memory):
    x = input_bits.astype(jnp.bfloat16)
    flat_c = (jnp.arange(_N, dtype=jnp.int32)[:, None] * _C
              + connections).reshape(_N * _NBITS)
    pw = jnp.broadcast_to(
        (1 << jnp.arange(_NBITS)).astype(jnp.float32)[None, :],
        (_N, _NBITS)).reshape(_N * _NBITS)
    w = (jnp.zeros((_N * _C,), jnp.float32).at[flat_c].add(pw)
         .reshape(_N, _C))

    idx = _addr_call(x, w)
    idx_flat = (idx.reshape(_N // 8, 8, _B // 128, 128)
                .transpose(0, 2, 1, 3).reshape(_TOT))
    mem_flat = (memory.reshape(_N // 8, 8, _M // 128, 128)
                .transpose(0, 2, 1, 3).reshape(_N * _M))
    vals = _lookup_call(idx_flat, mem_flat)
    out = (vals.reshape(_N // 8, _B // 128, 8, 128)
           .transpose(0, 2, 1, 3).reshape(_N, _B))
    return out.T.astype(jnp.bool_)


# speedup vs baseline: 1.0472x; 1.0472x over previous
import functools

import jax
import jax.numpy as jnp
from jax import lax
from jax.experimental import pallas as pl
from jax.experimental.pallas import tpu as pltpu
from jax.experimental.pallas import tpu_sc as plsc

_B = 512
_C = 2048
_N = 4096
_NBITS = 14
_M = 1 << _NBITS

_BN = 512

_NW = 32
_TOT = _B * _N
_PW = _TOT // _NW
_CH = 8192
_NCH = _PW // _CH
_SUB = 128
_NSUB = _CH // _SUB


def _addr_body(x_ref, w_ref, out_ref):
    w = w_ref[...]
    whi = jnp.floor(w * (1.0 / 128.0))
    wlo = w - whi * 128.0
    dims = (((1,), (1,)), ((), ()))
    lo = lax.dot_general(wlo.astype(jnp.bfloat16), x_ref[...], dims,
                         preferred_element_type=jnp.float32)
    hi = lax.dot_general(whi.astype(jnp.bfloat16), x_ref[...], dims,
                         preferred_element_type=jnp.float32)
    n0 = pl.program_id(0) * _BN
    n = lax.broadcasted_iota(jnp.int32, lo.shape, 0) + n0
    a = lo.astype(jnp.int32) + (hi.astype(jnp.int32) << 7)
    out_ref[...] = (((n >> 3) << 17) + ((a >> 7) << 10)
                    + ((n & 7) << 7) + (a & 127))


_addr_call = pl.pallas_call(
    _addr_body,
    grid=(_N // _BN,),
    in_specs=[
        pl.BlockSpec((_B, _C), lambda j: (0, 0)),
        pl.BlockSpec((_BN, _C), lambda j: (j, 0)),
    ],
    out_specs=pl.BlockSpec((_BN, _B), lambda j: (j, 0)),
    out_shape=jax.ShapeDtypeStruct((_N, _B), jnp.int32),
)


def _lookup_body(idx_hbm, mem_hbm, out_hbm, idx0, idx1, val0, val1,
                 sem0, sem1):
    wid = lax.axis_index("s") * 2 + lax.axis_index("c")
    base = wid * _PW

    def fire(i, idx_v, val_v, sem):
        o0 = base + i * _CH
        pltpu.sync_copy(idx_hbm.at[pl.ds(o0, _CH)], idx_v)
        for k in range(_NSUB):
            pltpu.async_copy(
                mem_hbm.at[idx_v.at[pl.ds(k * _SUB, _SUB)]],
                val_v.at[pl.ds(k * _SUB, _SUB)],
                sem,
            )

    def finish(i, val_v, sem):
        pltpu.make_async_copy(mem_hbm.at[pl.ds(0, _CH)], val_v, sem).wait()

        def inner(j, c):
            v = val_v[pl.ds(j * 16, 16)]
            one = jnp.full((16,), 1, jnp.int32)
            zero = jnp.zeros((16,), jnp.int32)
            val_v[pl.ds(j * 16, 16)] = jnp.where(v == one, one, zero)
            return c

        lax.fori_loop(0, _CH // 16, inner, 0, unroll=8)
        pltpu.sync_copy(val_v, out_hbm.at[pl.ds(base + i * _CH, _CH)])

    fire(0, idx0, val0, sem0)

    def pair(h, carry):
        i0 = 2 * h
        fire(i0 + 1, idx1, val1, sem1)
        finish(i0, val0, sem0)
        fire(i0 + 2, idx0, val0, sem0)
        finish(i0 + 1, val1, sem1)
        return carry

    lax.fori_loop(0, _NCH // 2 - 1, pair, 0)
    fire(_NCH - 1, idx1, val1, sem1)
    finish(_NCH - 2, val0, sem0)
    finish(_NCH - 1, val1, sem1)


_lookup_call = functools.partial(
    pl.kernel,
    mesh=plsc.VectorSubcoreMesh(core_axis_name="c", subcore_axis_name="s"),
    out_type=jax.ShapeDtypeStruct((_TOT,), jnp.int32),
    scratch_types=[
        pltpu.VMEM((_CH,), jnp.int32),
        pltpu.VMEM((_CH,), jnp.int32),
        pltpu.VMEM((_CH,), jnp.int32),
        pltpu.VMEM((_CH,), jnp.int32),
        pltpu.SemaphoreType.DMA,
        pltpu.SemaphoreType.DMA,
    ],
)(_lookup_body)


def kernel(input_bits, connections, memory):
    x = input_bits.astype(jnp.bfloat16)
    flat_c = (jnp.arange(_N, dtype=jnp.int32)[:, None] * _C
              + connections).reshape(_N * _NBITS)
    pw = jnp.broadcast_to(
        (1 << jnp.arange(_NBITS)).astype(jnp.float32)[None, :],
        (_N, _NBITS)).reshape(_N * _NBITS)
    w = (jnp.zeros((_N * _C,), jnp.float32).at[flat_c].add(pw)
         .reshape(_N, _C))

    idx = _addr_call(x, w)
    idx_flat = (idx.reshape(_N // 8, 8, _B // 128, 128)
                .transpose(0, 2, 1, 3).reshape(_TOT))
    mem_flat = (memory.reshape(_N // 8, 8, _M // 128, 128)
                .transpose(0, 2, 1, 3).reshape(_N * _M))
    vals = _lookup_call(idx_flat, mem_flat)
    out = (vals.reshape(_N // 8, _B // 128, 8, 128)
           .transpose(0, 2, 1, 3).reshape(_N, _B))
    return out.T.astype(jnp.bool_)
